# Initial kernel scaffold; baseline (speedup 1.0000x reference)
import jax
import jax.numpy as jnp
from jax.experimental import pallas as pl


def kernel(x, edge_index, batch, W1, b1, W2, b2, W3, b3, Wf1, bf1, Wf2, bf2):
    raise NotImplementedError("write your pallas kernel here")


# speedup vs baseline: 14.2481x; 14.2481x over previous
import functools

import jax
import jax.numpy as jnp
from jax import lax
from jax.experimental import pallas as pl
from jax.experimental.pallas import tpu as pltpu
from jax.experimental.pallas import tpu_sc as plsc

NN = 50000
EE = 800000
GG = 64
C = 128
NBLK = EE // C
PAD_N = 50048
STRIPE = 3128
ROW_BLK = 2000
NROW_BLK = NN // ROW_BLK

@functools.cache
def _mesh():
    return plsc.VectorSubcoreMesh(core_axis_name="c", subcore_axis_name="s",
                                  num_cores=2, num_subcores=16)


def _zero_acc(zeros_hbm, buf, acc, s):
    pltpu.sync_copy(zeros_hbm, buf)
    base = s * STRIPE
    for j in range(24):
        pltpu.sync_copy(buf, acc.at[pl.ds(base + j * C, C)])
    pltpu.sync_copy(buf.at[pl.ds(0, 56)], acc.at[pl.ds(base + 24 * C, 56)])


def _writeback(acc, buf, out_hbm, s, c=None):
    base = s * STRIPE

    def _dst(off, n):
        return (out_hbm.at[pl.ds(off, n)] if c is None
                else out_hbm.at[c, pl.ds(off, n)])

    for j in range(24):
        pltpu.sync_copy(acc.at[pl.ds(base + j * C, C)], buf)
        pltpu.sync_copy(buf, _dst(base + j * C, C))
    pltpu.sync_copy(acc.at[pl.ds(base + 24 * C, 56)], buf.at[pl.ds(0, 56)])
    pltpu.sync_copy(buf.at[pl.ds(0, 56)], _dst(base + 24 * C, 56))


@functools.cache
def _make_sc_deg():
    @functools.partial(
        pl.kernel, mesh=_mesh(),
        out_type=[jax.ShapeDtypeStruct((PAD_N,), jnp.float32),
                  jax.ShapeDtypeStruct((PAD_N,), jnp.float32)],
        scratch_types=[
            pltpu.VMEM((C,), jnp.int32),
            pltpu.VMEM((C,), jnp.float32),
            pltpu.VMEM_SHARED((PAD_N,), jnp.float32),
        ],
    )
    def k(dst_hbm, zeros_hbm, out0_hbm, out1_hbm, dsti, ones_v, acc):
        c = lax.axis_index("c")
        s = lax.axis_index("s")
        wid = c * 16 + s
        _zero_acc(zeros_hbm, ones_v, acc, s)
        for j in range(C // 16):
            ones_v[pl.ds(j * 16, 16)] = jnp.ones((16,), jnp.float32)
        plsc.subcore_barrier()
        base, rem = NBLK // 32, NBLK % 32
        nb = base + (wid < rem).astype(jnp.int32)
        start = wid * base + jnp.minimum(wid, rem)

        def body(b, carry):
            eoff = (start + b) * C
            pltpu.sync_copy(dst_hbm.at[pl.ds(eoff, C)], dsti)
            pltpu.sync_copy(ones_v, acc.at[dsti], add=True)
            return carry

        lax.fori_loop(0, nb, body, 0)
        plsc.subcore_barrier()

        @pl.when(c == 0)
        def _w0():
            _writeback(acc, ones_v, out0_hbm, s)

        @pl.when(c == 1)
        def _w1():
            _writeback(acc, ones_v, out1_hbm, s)

    return k


@functools.cache
def _make_sc_prop(W, feature_split):
    n_workers = 16 if feature_split else 32
    base, rem = NBLK // n_workers, NBLK % n_workers

    @functools.partial(
        pl.kernel, mesh=_mesh(),
        compiler_params=pltpu.CompilerParams(use_tc_tiling_on_sc=False),
        out_type=jax.ShapeDtypeStruct((2, PAD_N, W), jnp.float32),
        scratch_types=[
            pltpu.VMEM((C,), jnp.int32),
            pltpu.VMEM((C,), jnp.int32),
            pltpu.VMEM((C, W), jnp.float32),
            pltpu.VMEM_SHARED((PAD_N, W), jnp.float32),
            pltpu.SemaphoreType.DMA,
        ],
    )
    def k(src_hbm, dst_hbm, g_hbm, zeros_hbm, out_hbm,
          dsti, srci, rows, acc, sem):
        c = lax.axis_index("c")
        s = lax.axis_index("s")
        wid = s if feature_split else c * 16 + s
        _zero_acc(zeros_hbm, rows, acc, s)
        plsc.subcore_barrier()
        nb = base + (wid < rem).astype(jnp.int32)
        start = wid * base + jnp.minimum(wid, rem)

        def body(b, carry):
            eoff = (start + b) * C
            pltpu.sync_copy(dst_hbm.at[pl.ds(eoff, C)], dsti)
            pltpu.sync_copy(src_hbm.at[pl.ds(eoff, C)], srci)
            if feature_split:
                off = c * NN
                for j in range(C // 16):
                    sl = pl.ds(j * 16, 16)
                    srci[sl] = srci[sl] + off
            pltpu.async_copy(g_hbm.at[srci], rows, sem).wait()
            pltpu.sync_copy(rows, acc.at[dsti], add=True)
            return carry

        lax.fori_loop(0, nb, body, 0)
        plsc.subcore_barrier()
        _writeback(acc, rows, out_hbm, s, c=c)

    return k


def _row_spec(width, rank3_lead=None):
    if rank3_lead is None:
        return pl.BlockSpec((ROW_BLK, width), lambda i: (i, 0))
    return pl.BlockSpec((rank3_lead, ROW_BLK, width), lambda i: (0, i, 0))


def _full_spec(shape):
    nd = len(shape)
    return pl.BlockSpec(shape, lambda i, _n=nd: (0,) * _n)


def _tc_prelude(d0, d1, xp):
    def body(d0_ref, d1_ref, x_ref, dinv_ref, g1_ref):
        deg = d0_ref[...] + d1_ref[...] + 1.0
        dv = lax.rsqrt(deg)
        dinv_ref[...] = dv
        g1_ref[...] = x_ref[...] * dv

    return pl.pallas_call(
        body,
        grid=(NROW_BLK,),
        in_specs=[_row_spec(1), _row_spec(1), _row_spec(4)],
        out_specs=[_row_spec(1), _row_spec(4)],
        out_shape=[
            jax.ShapeDtypeStruct((NN, 1), jnp.float32),
            jax.ShapeDtypeStruct((NN, 4), jnp.float32),
        ],
    )(d0, d1, xp)


def _tc_layer(s_parts, g, dinv, Wm, bias, *, split_out):
    win = g.shape[1]
    fout = Wm.shape[1]

    def body(s_ref, g_ref, dinv_ref, w_ref, b_ref, out_ref):
        dv = dinv_ref[...]
        pre = dv * (s_ref[0] + s_ref[1] + g_ref[...])
        h = jnp.maximum(
            jnp.dot(pre, w_ref[...], preferred_element_type=jnp.float32)
            + b_ref[...], 0.0)
        gn = dv * h
        if split_out:
            out_ref[0] = gn[:, : fout // 2]
            out_ref[1] = gn[:, fout // 2:]
        else:
            out_ref[...] = gn

    out_spec = _row_spec(fout // 2, 2) if split_out else _row_spec(fout)
    out_shape = ((2, NN, fout // 2) if split_out else (NN, fout))
    return pl.pallas_call(
        body,
        grid=(NROW_BLK,),
        in_specs=[_row_spec(win, 2), _row_spec(win), _row_spec(1),
                  _full_spec(Wm.shape), _full_spec(bias.shape)],
        out_specs=out_spec,
        out_shape=jax.ShapeDtypeStruct(out_shape, jnp.float32),
    )(s_parts, g, dinv, Wm, bias)


def _tc_final(s3, g3, dinv, W3m, b3m, batch3, Wf1m, bf1m, Wf2m, bf2m):
    def body(s_ref, g_ref, dinv_ref, w3_ref, b3_ref, batch_ref,
             wf1_ref, bf1_ref, wf2_ref, bf2_ref, out_ref,
             pool_acc, cnt_acc):
        i = pl.program_id(0)

        @pl.when(i == 0)
        def _init():
            pool_acc[...] = jnp.zeros_like(pool_acc)
            cnt_acc[...] = jnp.zeros_like(cnt_acc)

        scat = jnp.concatenate([s_ref[0], s_ref[1]], axis=1)
        gcat = jnp.concatenate([g_ref[0], g_ref[1]], axis=1)
        dv = dinv_ref[...]
        pre = dv * (scat + gcat)
        h3 = jnp.maximum(
            jnp.dot(pre, w3_ref[...], preferred_element_type=jnp.float32)
            + b3_ref[...], 0.0)
        bvec = batch_ref[0, 0, :]
        iota = lax.broadcasted_iota(jnp.int32, (GG, ROW_BLK), 0)
        onehot = (iota == bvec[None, :]).astype(jnp.float32)
        pool_acc[...] += jnp.dot(onehot, h3,
                                 preferred_element_type=jnp.float32)
        cnt_acc[...] += jnp.sum(onehot, axis=1, keepdims=True)

        @pl.when(i == NROW_BLK - 1)
        def _fin():
            pooled = pool_acc[...] / jnp.maximum(cnt_acc[...], 1.0)
            h4 = jnp.maximum(
                jnp.dot(pooled, wf1_ref[...],
                        preferred_element_type=jnp.float32) + bf1_ref[...],
                0.0)
            out_ref[...] = (
                jnp.dot(h4, wf2_ref[...], preferred_element_type=jnp.float32)
                + bf2_ref[...])

    return pl.pallas_call(
        body,
        grid=(NROW_BLK,),
        in_specs=[_row_spec(32, 2), _row_spec(32, 2), _row_spec(1),
                  _full_spec(W3m.shape), _full_spec(b3m.shape),
                  pl.BlockSpec((1, 1, ROW_BLK), lambda i: (i, 0, 0)),
                  _full_spec(Wf1m.shape), _full_spec(bf1m.shape),
                  _full_spec(Wf2m.shape), _full_spec(bf2m.shape)],
        out_specs=_full_spec((GG, 10)),
        out_shape=jax.ShapeDtypeStruct((GG, 10), jnp.float32),
        scratch_shapes=[pltpu.VMEM((GG, 128), jnp.float32),
                        pltpu.VMEM((GG, 1), jnp.float32)],
    )(s3, g3, dinv, W3m, b3m, batch3, Wf1m, bf1m, Wf2m, bf2m)


def kernel(x, edge_index, batch, W1, b1, W2, b2, W3, b3, Wf1, bf1, Wf2, bf2):
    src = edge_index[0]
    dst = edge_index[1]
    xp = jnp.pad(x.reshape(-1, 3), ((0, 0), (0, 1)))
    W1p = jnp.pad(W1, ((0, 1), (0, 0)))
    z4 = jnp.zeros((C, 4), jnp.float32)
    z32 = jnp.zeros((C, 32), jnp.float32)

    d0, d1 = _make_sc_deg()(dst, jnp.zeros((C,), jnp.float32))
    dinv, g1 = _tc_prelude(d0.reshape(PAD_N, 1), d1.reshape(PAD_N, 1), xp)
    s1 = _make_sc_prop(4, False)(src, dst, g1, z4)
    g2 = _tc_layer(s1, g1, dinv, W1p, b1.reshape(1, -1),
                   split_out=False)
    s2 = _make_sc_prop(32, False)(src, dst, g2, z32)
    g3 = _tc_layer(s2, g2, dinv, W2, b2.reshape(1, -1),
                   split_out=True)
    s3 = _make_sc_prop(32, True)(src, dst, g3.reshape(2 * NN, 32), z32)
    out = _tc_final(s3, g3, dinv, W3, b3.reshape(1, -1),
                    batch.reshape(NROW_BLK, 1, ROW_BLK),
                    Wf1, bf1.reshape(1, -1), Wf2, bf2.reshape(1, -1))
    return out


# speedup vs baseline: 27.4198x; 1.9245x over previous
import functools

import jax
import jax.numpy as jnp
from jax import lax
from jax.experimental import pallas as pl
from jax.experimental.pallas import tpu as pltpu
from jax.experimental.pallas import tpu_sc as plsc

NN = 50000
EE = 800000
GG = 64
C = 128
SUP = 8
NCHUNK = 6256
SUPS = NCHUNK // SUP
PAD_N = 50048
STRIPE = 3128
ROW_BLK = 2000
NROW_BLK = NN // ROW_BLK

@functools.cache
def _mesh():
    return plsc.VectorSubcoreMesh(core_axis_name="c", subcore_axis_name="s",
                                  num_cores=2, num_subcores=16)


def _zero_acc(zeros_hbm, buf, acc, s):
    pltpu.sync_copy(zeros_hbm, buf)
    base = s * STRIPE
    for j in range(24):
        pltpu.sync_copy(buf, acc.at[pl.ds(base + j * C, C)])
    pltpu.sync_copy(buf.at[pl.ds(0, 56)], acc.at[pl.ds(base + 24 * C, 56)])


def _writeback(acc, buf, out_hbm, s, c=None):
    base = s * STRIPE

    def _dst(off, n):
        return (out_hbm.at[pl.ds(off, n)] if c is None
                else out_hbm.at[c, pl.ds(off, n)])

    for j in range(24):
        pltpu.sync_copy(acc.at[pl.ds(base + j * C, C)], buf)
        pltpu.sync_copy(buf, _dst(base + j * C, C))
    pltpu.sync_copy(acc.at[pl.ds(base + 24 * C, 56)], buf.at[pl.ds(0, 56)])
    pltpu.sync_copy(buf.at[pl.ds(0, 56)], _dst(base + 24 * C, 56))


@functools.cache
def _make_sc_deg():
    @functools.partial(
        pl.kernel, mesh=_mesh(),
        out_type=[jax.ShapeDtypeStruct((PAD_N,), jnp.float32),
                  jax.ShapeDtypeStruct((PAD_N,), jnp.float32)],
        scratch_types=[
            pltpu.VMEM((SUP, C), jnp.int32),
            pltpu.VMEM((C,), jnp.float32),
            pltpu.VMEM_SHARED((PAD_N,), jnp.float32),
        ],
    )
    def k(dst_hbm, zeros_hbm, out0_hbm, out1_hbm, dstb, ones_v, acc):
        c = lax.axis_index("c")
        s = lax.axis_index("s")
        wid = c * 16 + s
        _zero_acc(zeros_hbm, ones_v, acc, s)
        for j in range(C // 16):
            ones_v[pl.ds(j * 16, 16)] = jnp.ones((16,), jnp.float32)
        plsc.subcore_barrier()
        base, rem = SUPS // 32, SUPS % 32
        nsup = base + (wid < rem).astype(jnp.int32)
        start = wid * base + jnp.minimum(wid, rem)

        def body(t, carry):
            sup = start + t
            pltpu.sync_copy(dst_hbm.at[pl.ds(sup * SUP, SUP)], dstb)
            for j in range(SUP):
                pltpu.sync_copy(ones_v, acc.at[dstb.at[j]], add=True)
            return carry

        lax.fori_loop(0, nsup, body, 0)
        plsc.subcore_barrier()

        @pl.when(c == 0)
        def _w0():
            _writeback(acc, ones_v, out0_hbm, s)

        @pl.when(c == 1)
        def _w1():
            _writeback(acc, ones_v, out1_hbm, s)

    return k


@functools.cache
def _make_sc_prop(W, feature_split):
    n_workers = 16 if feature_split else 32
    base, rem = SUPS // n_workers, SUPS % n_workers

    @functools.partial(
        pl.kernel, mesh=_mesh(),
        compiler_params=pltpu.CompilerParams(use_tc_tiling_on_sc=False),
        out_type=jax.ShapeDtypeStruct((2, PAD_N, W), jnp.float32),
        scratch_types=[
            pltpu.VMEM((SUP, C), jnp.int32),
            pltpu.VMEM((SUP, C), jnp.int32),
            pltpu.VMEM((C, W), jnp.float32),
            pltpu.VMEM((C, W), jnp.float32),
            pltpu.VMEM_SHARED((PAD_N, W), jnp.float32),
            pltpu.SemaphoreType.DMA,
            pltpu.SemaphoreType.DMA,
        ],
    )
    def k(src_hbm, dst_hbm, g_hbm, zeros_hbm, out_hbm,
          dstb, srcb, rows0, rows1, acc, sg0, sg1):
        c = lax.axis_index("c")
        s = lax.axis_index("s")
        wid = s if feature_split else c * 16 + s
        _zero_acc(zeros_hbm, rows0, acc, s)
        plsc.subcore_barrier()
        nsup = base + (wid < rem).astype(jnp.int32)
        start = wid * base + jnp.minimum(wid, rem)
        idx_base = c * NCHUNK if feature_split else 0
        rows = (rows0, rows1)
        sems = (sg0, sg1)

        def body(t, carry):
            sup = start + t
            pltpu.sync_copy(src_hbm.at[pl.ds(idx_base + sup * SUP, SUP)], srcb)
            pltpu.sync_copy(dst_hbm.at[pl.ds(sup * SUP, SUP)], dstb)
            hs = [None] * SUP
            hs[0] = pltpu.async_copy(g_hbm.at[srcb.at[0]], rows[0], sems[0])
            for j in range(SUP):
                if j + 1 < SUP:
                    hs[j + 1] = pltpu.async_copy(
                        g_hbm.at[srcb.at[j + 1]], rows[(j + 1) % 2],
                        sems[(j + 1) % 2])
                hs[j].wait()
                pltpu.sync_copy(rows[j % 2], acc.at[dstb.at[j]], add=True)
            return carry

        lax.fori_loop(0, nsup, body, 0)
        plsc.subcore_barrier()
        _writeback(acc, rows0, out_hbm, s, c=c)

    return k


def _row_spec(width, rank3_lead=None):
    if rank3_lead is None:
        return pl.BlockSpec((ROW_BLK, width), lambda i: (i, 0))
    return pl.BlockSpec((rank3_lead, ROW_BLK, width), lambda i: (0, i, 0))


def _full_spec(shape):
    nd = len(shape)
    return pl.BlockSpec(shape, lambda i, _n=nd: (0,) * _n)


def _tc_prelude(d0, d1, xp):
    def body(d0_ref, d1_ref, x_ref, dinv_ref, g1_ref):
        deg = d0_ref[...] + d1_ref[...] + 1.0
        dv = lax.rsqrt(deg)
        dinv_ref[...] = dv
        g1_ref[...] = x_ref[...] * dv

    return pl.pallas_call(
        body,
        grid=(NROW_BLK,),
        in_specs=[_row_spec(1), _row_spec(1), _row_spec(4)],
        out_specs=[_row_spec(1), _row_spec(4)],
        out_shape=[
            jax.ShapeDtypeStruct((NN, 1), jnp.float32),
            jax.ShapeDtypeStruct((NN, 4), jnp.float32),
        ],
    )(d0, d1, xp)


def _tc_layer(s_parts, g, dinv, Wm, bias, *, split_out):
    win = g.shape[1]
    fout = Wm.shape[1]

    def body(s_ref, g_ref, dinv_ref, w_ref, b_ref, out_ref):
        dv = dinv_ref[...]
        pre = dv * (s_ref[0] + s_ref[1] + g_ref[...])
        h = jnp.maximum(
            jnp.dot(pre, w_ref[...], preferred_element_type=jnp.float32)
            + b_ref[...], 0.0)
        gn = dv * h
        if split_out:
            out_ref[0] = gn[:, : fout // 2]
            out_ref[1] = gn[:, fout // 2:]
        else:
            out_ref[...] = gn

    out_spec = _row_spec(fout // 2, 2) if split_out else _row_spec(fout)
    out_shape = ((2, NN, fout // 2) if split_out else (NN, fout))
    return pl.pallas_call(
        body,
        grid=(NROW_BLK,),
        in_specs=[_row_spec(win, 2), _row_spec(win), _row_spec(1),
                  _full_spec(Wm.shape), _full_spec(bias.shape)],
        out_specs=out_spec,
        out_shape=jax.ShapeDtypeStruct(out_shape, jnp.float32),
    )(s_parts, g, dinv, Wm, bias)


def _tc_final(s3, g3, dinv, W3m, b3m, batch3, Wf1m, bf1m, Wf2m, bf2m):
    def body(s_ref, g_ref, dinv_ref, w3_ref, b3_ref, batch_ref,
             wf1_ref, bf1_ref, wf2_ref, bf2_ref, out_ref,
             pool_acc, cnt_acc):
        i = pl.program_id(0)

        @pl.when(i == 0)
        def _init():
            pool_acc[...] = jnp.zeros_like(pool_acc)
            cnt_acc[...] = jnp.zeros_like(cnt_acc)

        scat = jnp.concatenate([s_ref[0], s_ref[1]], axis=1)
        gcat = jnp.concatenate([g_ref[0], g_ref[1]], axis=1)
        dv = dinv_ref[...]
        pre = dv * (scat + gcat)
        h3 = jnp.maximum(
            jnp.dot(pre, w3_ref[...], preferred_element_type=jnp.float32)
            + b3_ref[...], 0.0)
        bvec = batch_ref[0, 0, :]
        iota = lax.broadcasted_iota(jnp.int32, (GG, ROW_BLK), 0)
        onehot = (iota == bvec[None, :]).astype(jnp.float32)
        pool_acc[...] += jnp.dot(onehot, h3,
                                 preferred_element_type=jnp.float32)
        cnt_acc[...] += jnp.sum(onehot, axis=1, keepdims=True)

        @pl.when(i == NROW_BLK - 1)
        def _fin():
            pooled = pool_acc[...] / jnp.maximum(cnt_acc[...], 1.0)
            h4 = jnp.maximum(
                jnp.dot(pooled, wf1_ref[...],
                        preferred_element_type=jnp.float32) + bf1_ref[...],
                0.0)
            out_ref[...] = (
                jnp.dot(h4, wf2_ref[...], preferred_element_type=jnp.float32)
                + bf2_ref[...])

    return pl.pallas_call(
        body,
        grid=(NROW_BLK,),
        in_specs=[_row_spec(32, 2), _row_spec(32, 2), _row_spec(1),
                  _full_spec(W3m.shape), _full_spec(b3m.shape),
                  pl.BlockSpec((1, 1, ROW_BLK), lambda i: (i, 0, 0)),
                  _full_spec(Wf1m.shape), _full_spec(bf1m.shape),
                  _full_spec(Wf2m.shape), _full_spec(bf2m.shape)],
        out_specs=_full_spec((GG, 10)),
        out_shape=jax.ShapeDtypeStruct((GG, 10), jnp.float32),
        scratch_shapes=[pltpu.VMEM((GG, 128), jnp.float32),
                        pltpu.VMEM((GG, 1), jnp.float32)],
    )(s3, g3, dinv, W3m, b3m, batch3, Wf1m, bf1m, Wf2m, bf2m)


def kernel(x, edge_index, batch, W1, b1, W2, b2, W3, b3, Wf1, bf1, Wf2, bf2):
    npad = NCHUNK * C - EE
    src2 = jnp.concatenate(
        [edge_index[0],
         jnp.arange(npad, dtype=jnp.int32) % NN]).reshape(NCHUNK, C)
    dst2 = jnp.concatenate(
        [edge_index[1],
         NN + jnp.arange(npad, dtype=jnp.int32) % (PAD_N - NN)]
    ).reshape(NCHUNK, C)
    src2fs = jnp.concatenate([src2, src2 + NN], axis=0)
    xp = jnp.pad(x.reshape(-1, 3), ((0, 0), (0, 1)))
    W1p = jnp.pad(W1, ((0, 1), (0, 0)))
    z4 = jnp.zeros((C, 4), jnp.float32)
    z32 = jnp.zeros((C, 32), jnp.float32)

    d0, d1 = _make_sc_deg()(dst2, jnp.zeros((C,), jnp.float32))
    dinv, g1 = _tc_prelude(d0.reshape(PAD_N, 1), d1.reshape(PAD_N, 1), xp)
    s1 = _make_sc_prop(4, False)(src2, dst2, g1, z4)
    g2 = _tc_layer(s1, g1, dinv, W1p, b1.reshape(1, -1),
                   split_out=False)
    s2 = _make_sc_prop(32, False)(src2, dst2, g2, z32)
    g3 = _tc_layer(s2, g2, dinv, W2, b2.reshape(1, -1),
                   split_out=True)
    s3 = _make_sc_prop(32, True)(src2fs, dst2, g3.reshape(2 * NN, 32), z32)
    out = _tc_final(s3, g3, dinv, W3, b3.reshape(1, -1),
                    batch.reshape(NROW_BLK, 1, ROW_BLK),
                    Wf1, bf1.reshape(1, -1), Wf2, bf2.reshape(1, -1))
    return out
